# baseline (speedup 1.0000x reference)
import jax
import jax.numpy as jnp
from jax.experimental import pallas as pl
from jax.experimental.pallas import tpu as pltpu

_NE = 8192
_D = 256
_K = 1024
_TB = 256
_COMMIT = 0.25


def _vq_body(feat_ref, w_ref, b_ref, emb_ref, embb_ref,
             idx_ref, qst_ref, loss_ref, esq_ref):
    i = pl.program_id(0)
    ntok = pl.num_programs(0) * _TB

    @pl.when(i == 0)
    def _():
        e = emb_ref[...]
        esq_ref[...] = jnp.sum(e * e, axis=1)[None, :]

    proj = jax.lax.dot_general(
        feat_ref[...], w_ref[...], (((1,), (1,)), ((), ())),
        preferred_element_type=jnp.float32) + b_ref[...]

    xsq = jnp.sum(proj * proj, axis=1, keepdims=True)

    m = jax.lax.dot_general(
        proj.astype(jnp.bfloat16), embb_ref[...], (((1,), (1,)), ((), ())),
        preferred_element_type=jnp.float32)
    d = (xsq + esq_ref[...]) - 2.0 * m

    dmin = jnp.min(d, axis=1, keepdims=True)
    iota = jax.lax.broadcasted_iota(jnp.int32, d.shape, 1)
    idx = jnp.min(jnp.where(d == dmin, iota, _NE), axis=1)
    idx_ref[0, 0, :] = idx

    onehot = (iota == idx[:, None]).astype(jnp.bfloat16)
    q = jax.lax.dot_general(
        onehot, embb_ref[...], (((1,), (0,)), ((), ())),
        preferred_element_type=jnp.float32)
    qst_ref[...] = proj + (q - proj)

    @pl.when(i == 0)
    def _():
        loss_ref[...] = jnp.zeros_like(loss_ref)
    loss_ref[...] += jnp.sum(dmin).reshape(1, 1)

    @pl.when(i == pl.num_programs(0) - 1)
    def _():
        loss_ref[...] *= (1.0 + _COMMIT) / (ntok * _D)


def kernel(features, W_proj, b_proj, emb):
    B, T, _ = features.shape
    ntok = B * T
    grid = ntok // _TB
    featf = features.reshape(ntok, _K).astype(jnp.bfloat16)
    wb = W_proj.astype(jnp.bfloat16)
    embb = emb.astype(jnp.bfloat16)
    b2 = b_proj.reshape(1, _D)
    idx3, qst, loss = pl.pallas_call(
        _vq_body,
        grid=(grid,),
        in_specs=[
            pl.BlockSpec((_TB, _K), lambda i: (i, 0)),
            pl.BlockSpec((_D, _K), lambda i: (0, 0)),
            pl.BlockSpec((1, _D), lambda i: (0, 0)),
            pl.BlockSpec((_NE, _D), lambda i: (0, 0)),
            pl.BlockSpec((_NE, _D), lambda i: (0, 0)),
        ],
        out_specs=[
            pl.BlockSpec((1, 1, _TB), lambda i: (i, 0, 0)),
            pl.BlockSpec((_TB, _D), lambda i: (i, 0)),
            pl.BlockSpec((1, 1), lambda i: (0, 0)),
        ],
        out_shape=[
            jax.ShapeDtypeStruct((grid, 1, _TB), jnp.int32),
            jax.ShapeDtypeStruct((ntok, _D), jnp.float32),
            jax.ShapeDtypeStruct((1, 1), jnp.float32),
        ],
        scratch_shapes=[pltpu.VMEM((1, _NE), jnp.float32)],
    )(featf, wb, b2, emb, embb)
    quantized_st = qst.reshape(B, T, _D)
    return quantized_st, loss[0, 0], idx3.reshape(B, T)
